# Initial kernel scaffold; baseline (speedup 1.0000x reference)
import jax
import jax.numpy as jnp
from jax.experimental import pallas as pl


def kernel(x, edge_index, batch, y, W1, b1, Ws, bs, W2, b2, fc1_W, fc1_b, fc2_W, fc2_b, fc3_W, fc3_b, fc4_W, fc4_b, fc5_W, fc5_b, ct_vec, cs_vec, al_vec, cet_vec):
    raise NotImplementedError("write your pallas kernel here")


# speedup vs baseline: 1.0012x; 1.0012x over previous
import jax
import jax.numpy as jnp
from jax.experimental import pallas as pl
from jax.experimental.pallas import tpu as pltpu

N = 50000
E = 1600000
K = 35000


def _gcn_conv(x, src, dst, W, b, mask):
    n = x.shape[0]
    xw = x @ W
    ew = mask[src] * mask[dst]
    deg = jnp.zeros((n,), xw.dtype).at[dst].add(ew) + mask
    dinv = jnp.where(deg > 0, 1.0 / jnp.sqrt(jnp.maximum(deg, 1e-12)), 0.0)
    norm = dinv[src] * dinv[dst] * ew
    agg = jnp.zeros_like(xw).at[dst].add(norm[:, None] * xw[src])
    agg = agg + (dinv * dinv * mask)[:, None] * xw
    return agg + b


def _final_scale_kernel(t32_ref, fc1W_ref, fc1b_ref, c_ref, o_ref):
    t = t32_ref[0, :]
    w = fc1W_ref[0, :]
    csum = jnp.sum(c_ref[0, :])
    m = (jnp.sum(t * w) + K * fc1b_ref[0, 0]) * csum / (4.0 * K)
    o_ref[...] = jnp.full((1, 1), m, jnp.float32)


def kernel(x, edge_index, batch, y, W1, b1, Ws, bs, W2, b2, fc1_W, fc1_b,
           fc2_W, fc2_b, fc3_W, fc3_b, fc4_W, fc4_b, fc5_W, fc5_b,
           ct_vec, cs_vec, al_vec, cet_vec):
    src, dst = edge_index[0], edge_index[1]
    ones = jnp.ones((N,), jnp.float32)
    h1 = jax.nn.relu(_gcn_conv(x, src, dst, W1, b1, ones))
    score = _gcn_conv(h1, src, dst, Ws, bs, ones)[:, 0]
    score_vals, perm = jax.lax.top_k(score, K)
    mask2 = jnp.zeros((N,), jnp.float32).at[perm].set(1.0)
    gate = jnp.where(mask2 > 0, jnp.tanh(score), 0.0)
    hf = h1 * gate[:, None]
    h2 = jax.nn.relu(_gcn_conv(hf, src, dst, W2, b2, mask2))
    t32 = jnp.sum(h2 * mask2[:, None], axis=0)

    c = jnp.concatenate([
        fc2_W @ ct_vec + fc2_b, fc3_W @ cs_vec + fc3_b,
        fc4_W @ al_vec + fc4_b, fc5_W @ cet_vec + fc5_b])

    out = pl.pallas_call(
        _final_scale_kernel,
        out_shape=jax.ShapeDtypeStruct((1, 1), jnp.float32),
    )(t32[None, :], fc1_W, fc1_b[None, :], c[None, :])
    return out[0]


# speedup vs baseline: 54.7472x; 54.6836x over previous
import functools

import jax
import jax.numpy as jnp
from jax import lax
from jax.experimental import pallas as pl
from jax.experimental.pallas import tpu as pltpu
from jax.experimental.pallas import tpu_sc as plsc

N = 50000
E = 1600000
K = 35000

NCORES = 2
NSUB = 16
NW = NCORES * NSUB
IDXW = 128
ROWS = E // IDXW
RPW = ROWS // NW
REM = ROWS - RPW * NW
CHUNK = 16
NFULL = RPW // CHUNK
NTAIL = RPW % CHUNK
ZR = 3200
ZR_LAST = N - 15 * ZR

NPAD = 50176
NROW = NPAD // 128


def _mesh():
    return plsc.VectorSubcoreMesh(core_axis_name="c", subcore_axis_name="s")


def _acc_zero(zero_hbm, acc, s):
    @pl.when(s < 15)
    def _():
        pltpu.sync_copy(zero_hbm.at[pl.ds(s * ZR, ZR)],
                        acc.at[pl.ds(s * ZR, ZR)])

    @pl.when(s == 15)
    def _():
        pltpu.sync_copy(zero_hbm.at[pl.ds(15 * ZR, ZR_LAST)],
                        acc.at[pl.ds(15 * ZR, ZR_LAST)])


def _acc_writeback(acc, out_hbm, c, s):
    @pl.when(s < 15)
    def _():
        pltpu.sync_copy(acc.at[pl.ds(s * ZR, ZR)],
                        out_hbm.at[c].at[pl.ds(s * ZR, ZR)])

    @pl.when(s == 15)
    def _():
        pltpu.sync_copy(acc.at[pl.ds(15 * ZR, ZR_LAST)],
                        out_hbm.at[c].at[pl.ds(15 * ZR, ZR_LAST)])


def _make_edge_agg(D):
    tshape = (N, D) if D else (N,)
    rshape = (IDXW, D) if D else (IDXW,)

    def body(u_hbm, src_hbm, dst_hbm, zero_hbm, out_hbm,
             sbuf, dbuf, rows, acc, sem):
        c = lax.axis_index("c")
        s = lax.axis_index("s")
        w = c * NSUB + s
        _acc_zero(zero_hbm, acc, s)
        plsc.subcore_barrier()

        def do_rows(row0, nrows):
            pltpu.sync_copy(src_hbm.at[pl.ds(row0, nrows)],
                            sbuf.at[pl.ds(0, nrows)])
            pltpu.sync_copy(dst_hbm.at[pl.ds(row0, nrows)],
                            dbuf.at[pl.ds(0, nrows)])

            def inner(k, carry):
                pltpu.async_copy(u_hbm.at[sbuf.at[k]], rows, sem).wait()
                pltpu.sync_copy(rows, acc.at[dbuf.at[k]], add=True)
                return carry

            lax.fori_loop(0, nrows, inner, 0)

        base = w * RPW

        def outer(j, carry):
            do_rows(base + j * CHUNK, CHUNK)
            return carry

        lax.fori_loop(0, NFULL, outer, 0)
        do_rows(base + NFULL * CHUNK, NTAIL)

        @pl.when(w < REM)
        def _():
            do_rows(NW * RPW + w, 1)

        plsc.subcore_barrier()
        _acc_writeback(acc, out_hbm, c, s)

    return pl.kernel(
        body,
        out_type=jax.ShapeDtypeStruct((NCORES,) + tshape, jnp.float32),
        mesh=_mesh(),
        compiler_params=pltpu.CompilerParams(use_tc_tiling_on_sc=False),
        scratch_types=[
            pltpu.VMEM((CHUNK, IDXW), jnp.int32),
            pltpu.VMEM((CHUNK, IDXW), jnp.int32),
            pltpu.VMEM(rshape, jnp.float32),
            pltpu.VMEM_SHARED(tshape, jnp.float32),
            pltpu.SemaphoreType.DMA,
        ],
    )


_EDGE_AGG_CACHE = {}


def _edge_agg(D):
    if D not in _EDGE_AGG_CACHE:
        _EDGE_AGG_CACHE[D] = _make_edge_agg(D)
    return _EDGE_AGG_CACHE[D]


def _agg8(u, src2d, dst2d, zeros):
    return _edge_agg(8)(u, src2d, dst2d, zeros)


def _agg32(u, src2d, dst2d, zeros):
    return _edge_agg(32)(u, src2d, dst2d, zeros)


def _agg_scalar(u, src2d, dst2d, zeros):
    return _edge_agg(0)(u, src2d, dst2d, zeros)


def _edge_count_body(dst_hbm, zero_hbm, out_hbm, dbuf, ones, acc):
    c = lax.axis_index("c")
    s = lax.axis_index("s")
    w = c * NSUB + s
    _acc_zero(zero_hbm, acc, s)

    def fill(k, carry):
        ones[pl.ds(k * 16, 16)] = jnp.ones((16,), jnp.float32)
        return carry

    lax.fori_loop(0, IDXW // 16, fill, 0)
    plsc.subcore_barrier()

    def do_rows(row0, nrows):
        pltpu.sync_copy(dst_hbm.at[pl.ds(row0, nrows)],
                        dbuf.at[pl.ds(0, nrows)])

        def inner(k, carry):
            pltpu.sync_copy(ones, acc.at[dbuf.at[k]], add=True)
            return carry

        lax.fori_loop(0, nrows, inner, 0)

    base = w * RPW

    def outer(j, carry):
        do_rows(base + j * CHUNK, CHUNK)
        return carry

    lax.fori_loop(0, NFULL, outer, 0)
    do_rows(base + NFULL * CHUNK, NTAIL)

    @pl.when(w < REM)
    def _():
        do_rows(NW * RPW + w, 1)

    plsc.subcore_barrier()
    _acc_writeback(acc, out_hbm, c, s)


def _edge_count(dst2d, zeros):
    if "count" not in _EDGE_AGG_CACHE:
        _EDGE_AGG_CACHE["count"] = pl.kernel(
            _edge_count_body,
            out_type=jax.ShapeDtypeStruct((NCORES, N), jnp.float32),
            mesh=_mesh(),
            compiler_params=pltpu.CompilerParams(use_tc_tiling_on_sc=False),
            scratch_types=[
                pltpu.VMEM((CHUNK, IDXW), jnp.int32),
                pltpu.VMEM((IDXW,), jnp.float32),
                pltpu.VMEM_SHARED((N,), jnp.float32),
            ],
        )
    return _EDGE_AGG_CACHE["count"](dst2d, zeros)


BLK = 1024
GRID = NPAD // BLK


def _tc_a_body(deg_ref, x_ref, dinv_ref, u1_ref):
    deg = deg_ref[0] + deg_ref[1] + 1.0
    dinv = lax.rsqrt(deg)
    dinv_ref[...] = dinv
    u1_ref[...] = dinv * x_ref[...]


def _tc_a(deg_planes, xpad):
    return pl.pallas_call(
        _tc_a_body,
        grid=(GRID,),
        in_specs=[
            pl.BlockSpec((2, BLK, 1), lambda i: (0, i, 0)),
            pl.BlockSpec((BLK, 8), lambda i: (i, 0)),
        ],
        out_specs=[
            pl.BlockSpec((BLK, 1), lambda i: (i, 0)),
            pl.BlockSpec((BLK, 8), lambda i: (i, 0)),
        ],
        out_shape=[
            jax.ShapeDtypeStruct((NPAD, 1), jnp.float32),
            jax.ShapeDtypeStruct((NPAD, 8), jnp.float32),
        ],
    )(deg_planes, xpad)


def _tc_b_body(agg_ref, x_ref, dinv_ref, W1_ref, b1_ref, Ws_ref,
               h1_ref, s_ref, su_ref):
    dinv = dinv_ref[...]
    agg = agg_ref[0] + agg_ref[1]
    z = dinv * agg + (dinv * dinv) * x_ref[...]
    h1 = jnp.maximum(jnp.dot(z, W1_ref[...],
                             preferred_element_type=jnp.float32)
                     + b1_ref[...], 0.0)
    h1_ref[...] = h1
    s = jnp.dot(h1, Ws_ref[...], preferred_element_type=jnp.float32)
    s_ref[...] = s
    su_ref[...] = dinv * s


def _tc_b(agg_planes, xpad, dinv, W1p, b1, Ws):
    return pl.pallas_call(
        _tc_b_body,
        grid=(GRID,),
        in_specs=[
            pl.BlockSpec((2, BLK, 8), lambda i: (0, i, 0)),
            pl.BlockSpec((BLK, 8), lambda i: (i, 0)),
            pl.BlockSpec((BLK, 1), lambda i: (i, 0)),
            pl.BlockSpec((8, 64), lambda i: (0, 0)),
            pl.BlockSpec((1, 64), lambda i: (0, 0)),
            pl.BlockSpec((64, 1), lambda i: (0, 0)),
        ],
        out_specs=[
            pl.BlockSpec((BLK, 64), lambda i: (i, 0)),
            pl.BlockSpec((BLK, 1), lambda i: (i, 0)),
            pl.BlockSpec((BLK, 1), lambda i: (i, 0)),
        ],
        out_shape=[
            jax.ShapeDtypeStruct((NPAD, 64), jnp.float32),
            jax.ShapeDtypeStruct((NPAD, 1), jnp.float32),
            jax.ShapeDtypeStruct((NPAD, 1), jnp.float32),
        ],
    )(agg_planes, xpad, dinv, W1p, b1, Ws)


def _tc_select_body(sagg_ref, s_ref, dinv_ref, bs_ref, mask_ref, gate_ref):
    dinv = dinv_ref[...]
    score = dinv * (sagg_ref[0] + sagg_ref[1]) + dinv * dinv * s_ref[...] \
        + bs_ref[0, 0]
    pos = (lax.broadcasted_iota(jnp.int32, (NROW, 128), 0) * 128
           + lax.broadcasted_iota(jnp.int32, (NROW, 128), 1))
    valid = pos < N
    bits = lax.bitcast_convert_type(score, jnp.uint32)
    key = jnp.where(bits >= jnp.uint32(0x80000000), ~bits,
                    bits | jnp.uint32(0x80000000))
    key = jnp.where(valid, key, jnp.uint32(0))

    def bit_step(i, t):
        b = jnp.uint32(31) - i.astype(jnp.uint32)
        cand = t | (jnp.uint32(1) << b)
        cnt = jnp.sum((key >= cand).astype(jnp.float32))
        return jnp.where(cnt >= K, cand, t)

    T = lax.fori_loop(0, 32, bit_step, jnp.uint32(0))
    n_gt = jnp.sum((key > T).astype(jnp.float32))
    need = K - n_gt
    eq = key == T

    def pos_step(i, xacc):
        b = 16 - i
        cand = xacc | (1 << b)
        cnt = jnp.sum(jnp.where(eq & (pos < cand), 1.0, 0.0))
        return jnp.where(cnt < need, cand, xacc)

    X = lax.fori_loop(0, 17, pos_step, 0)
    keep = (key > T) | (eq & (pos <= X) & (need >= 1.0))
    maskf = jnp.where(keep, 1.0, 0.0)
    mask_ref[...] = maskf
    gate_ref[...] = jnp.tanh(score) * maskf


def _tc_select(sagg_planes, s2d, dinv2d, bs):
    return pl.pallas_call(
        _tc_select_body,
        out_shape=[
            jax.ShapeDtypeStruct((NROW, 128), jnp.float32),
            jax.ShapeDtypeStruct((NROW, 128), jnp.float32),
        ],
    )(sagg_planes, s2d, dinv2d, bs)


def _tc_c2_body(h1_ref, gate_ref, W2_ref, hw2_ref):
    hf = h1_ref[...] * gate_ref[...]
    hw2_ref[...] = jnp.dot(hf, W2_ref[...],
                           preferred_element_type=jnp.float32)


def _tc_c2(h1, gate, W2):
    return pl.pallas_call(
        _tc_c2_body,
        grid=(GRID,),
        in_specs=[
            pl.BlockSpec((BLK, 64), lambda i: (i, 0)),
            pl.BlockSpec((BLK, 1), lambda i: (i, 0)),
            pl.BlockSpec((64, 32), lambda i: (0, 0)),
        ],
        out_specs=pl.BlockSpec((BLK, 32), lambda i: (i, 0)),
        out_shape=jax.ShapeDtypeStruct((NPAD, 32), jnp.float32),
    )(h1, gate, W2)


def _tc_d_body(magg_ref, mask_ref, hw2_ref, dinv2_ref, u2_ref):
    m = mask_ref[...]
    deg2 = m * (magg_ref[0] + magg_ref[1] + 1.0)
    dinv2 = jnp.where(deg2 > 0, 1.0 / jnp.sqrt(jnp.maximum(deg2, 1e-12)),
                      0.0)
    dinv2_ref[...] = dinv2
    u2_ref[...] = dinv2 * hw2_ref[...]


def _tc_d(magg_planes, mask, hw2):
    return pl.pallas_call(
        _tc_d_body,
        grid=(GRID,),
        in_specs=[
            pl.BlockSpec((2, BLK, 1), lambda i: (0, i, 0)),
            pl.BlockSpec((BLK, 1), lambda i: (i, 0)),
            pl.BlockSpec((BLK, 32), lambda i: (i, 0)),
        ],
        out_specs=[
            pl.BlockSpec((BLK, 1), lambda i: (i, 0)),
            pl.BlockSpec((BLK, 32), lambda i: (i, 0)),
        ],
        out_shape=[
            jax.ShapeDtypeStruct((NPAD, 1), jnp.float32),
            jax.ShapeDtypeStruct((NPAD, 32), jnp.float32),
        ],
    )(magg_planes, mask, hw2)


def _tc_e_body(agg2_ref, hw2_ref, dinv2_ref, mask_ref, b2_ref, t32_ref):
    i = pl.program_id(0)

    @pl.when(i == 0)
    def _():
        t32_ref[...] = jnp.zeros((1, 32), jnp.float32)

    dinv2 = dinv2_ref[...]
    agg2 = agg2_ref[0] + agg2_ref[1]
    h2 = jnp.maximum(dinv2 * agg2 + (dinv2 * dinv2) * hw2_ref[...]
                     + b2_ref[...], 0.0)
    t32_ref[...] += jnp.sum(h2 * mask_ref[...], axis=0, keepdims=True)


def _tc_e():
    return pl.pallas_call(
        _tc_e_body,
        grid=(GRID,),
        in_specs=[
            pl.BlockSpec((2, BLK, 32), lambda i: (0, i, 0)),
            pl.BlockSpec((BLK, 32), lambda i: (i, 0)),
            pl.BlockSpec((BLK, 1), lambda i: (i, 0)),
            pl.BlockSpec((BLK, 1), lambda i: (i, 0)),
            pl.BlockSpec((1, 32), lambda i: (0, 0)),
        ],
        out_specs=pl.BlockSpec((1, 32), lambda i: (0, 0)),
        out_shape=jax.ShapeDtypeStruct((1, 32), jnp.float32),
    )


def _tc_final_body(t32_ref, fc1W_ref, fc1b_ref, c_ref, o_ref):
    t = t32_ref[0, :]
    w = fc1W_ref[0, :]
    csum = jnp.sum(c_ref[0, :])
    m = (jnp.sum(t * w) + K * fc1b_ref[0, 0]) * csum / (4.0 * K)
    o_ref[...] = jnp.full((1, 1), m, jnp.float32)


def _padn(v):
    return jnp.pad(v, (0, NPAD - N))[:, None]


def kernel(x, edge_index, batch, y, W1, b1, Ws, bs, W2, b2, fc1_W, fc1_b,
           fc2_W, fc2_b, fc3_W, fc3_b, fc4_W, fc4_b, fc5_W, fc5_b,
           ct_vec, cs_vec, al_vec, cet_vec):
    src2d = edge_index[0].reshape(ROWS, IDXW)
    dst2d = edge_index[1].reshape(ROWS, IDXW)
    zeros1 = jnp.zeros((N,), jnp.float32)
    zeros8 = jnp.zeros((N, 8), jnp.float32)
    zeros32 = jnp.zeros((N, 32), jnp.float32)
    xpad = jnp.pad(x, ((0, NPAD - N), (0, 2)))
    W1p = jnp.pad(W1, ((0, 2), (0, 0)))

    deg_pl = _edge_count(dst2d, zeros1)
    deg_planes = jnp.pad(deg_pl, ((0, 0), (0, NPAD - N)))[:, :, None]
    dinv, u1 = _tc_a(deg_planes, xpad)

    agg1_pl = _agg8(u1[:N], src2d, dst2d, zeros8)
    agg1_planes = jnp.pad(agg1_pl, ((0, 0), (0, NPAD - N), (0, 0)))
    h1, s, su = _tc_b(agg1_planes, xpad, dinv, W1p, b1[None, :], Ws)

    sagg_pl = _agg_scalar(su[:N, 0], src2d, dst2d, zeros1)
    sagg2d = jnp.pad(sagg_pl, ((0, 0), (0, NPAD - N))).reshape(2, NROW, 128)
    mask2d, gate2d = _tc_select(sagg2d, s.reshape(NROW, 128),
                                dinv.reshape(NROW, 128), bs[None, :])
    mask = mask2d.reshape(NPAD, 1)
    gate = gate2d.reshape(NPAD, 1)

    hw2 = _tc_c2(h1, gate, W2)
    magg_pl = _agg_scalar(mask[:N, 0], src2d, dst2d, zeros1)
    magg_planes = jnp.pad(magg_pl, ((0, 0), (0, NPAD - N)))[:, :, None]
    dinv2, u2 = _tc_d(magg_planes, mask, hw2)
    agg2_pl = _agg32(u2[:N], src2d, dst2d, zeros32)
    agg2_planes = jnp.pad(agg2_pl, ((0, 0), (0, NPAD - N), (0, 0)))
    t32 = _tc_e()(agg2_planes, hw2, dinv2, mask, b2[None, :])

    c = jnp.concatenate([
        fc2_W @ ct_vec + fc2_b, fc3_W @ cs_vec + fc3_b,
        fc4_W @ al_vec + fc4_b, fc5_W @ cet_vec + fc5_b])
    out = pl.pallas_call(
        _tc_final_body,
        out_shape=jax.ShapeDtypeStruct((1, 1), jnp.float32),
    )(t32, fc1_W, fc1_b[None, :], c[None, :])
    return out[0]


# speedup vs baseline: 94.0981x; 1.7188x over previous
import functools

import jax
import jax.numpy as jnp
from jax import lax
from jax.experimental import pallas as pl
from jax.experimental.pallas import tpu as pltpu
from jax.experimental.pallas import tpu_sc as plsc

N = 50000
E = 1600000
K = 35000

NCORES = 2
NSUB = 16
NW = NCORES * NSUB
IDXW = 128
ROWS = E // IDXW
RPW = ROWS // NW
REM = ROWS - RPW * NW
CHUNK = 16
NFULL = RPW // CHUNK
NTAIL = RPW % CHUNK
ZR = 3200
ZR_LAST = N - 15 * ZR

NPAD = 50176
NROW = NPAD // 128


def _mesh():
    return plsc.VectorSubcoreMesh(core_axis_name="c", subcore_axis_name="s")


def _acc_zero(zero_hbm, acc, s):
    @pl.when(s < 15)
    def _():
        pltpu.sync_copy(zero_hbm.at[pl.ds(s * ZR, ZR)],
                        acc.at[pl.ds(s * ZR, ZR)])

    @pl.when(s == 15)
    def _():
        pltpu.sync_copy(zero_hbm.at[pl.ds(15 * ZR, ZR_LAST)],
                        acc.at[pl.ds(15 * ZR, ZR_LAST)])


def _acc_writeback(acc, out_hbm, c, s):
    @pl.when(s < 15)
    def _():
        pltpu.sync_copy(acc.at[pl.ds(s * ZR, ZR)],
                        out_hbm.at[c].at[pl.ds(s * ZR, ZR)])

    @pl.when(s == 15)
    def _():
        pltpu.sync_copy(acc.at[pl.ds(15 * ZR, ZR_LAST)],
                        out_hbm.at[c].at[pl.ds(15 * ZR, ZR_LAST)])


def _make_edge_agg(D):
    tshape = (N, D) if D else (N,)
    rshape = (IDXW, D) if D else (IDXW,)

    def body(u_hbm, src_hbm, dst_hbm, zero_hbm, out_hbm,
             sbuf, dbuf, rows0, rows1, acc, gs0, gs1, ss0, ss1):
        c = lax.axis_index("c")
        s = lax.axis_index("s")
        w = c * NSUB + s
        _acc_zero(zero_hbm, acc, s)
        plsc.subcore_barrier()
        rbufs = (rows0, rows1)
        gsems = (gs0, gs1)
        ssems = (ss0, ss1)

        def do_rows(row0, nrows):
            pltpu.sync_copy(src_hbm.at[pl.ds(row0, nrows)],
                            sbuf.at[pl.ds(0, nrows)])
            pltpu.sync_copy(dst_hbm.at[pl.ds(row0, nrows)],
                            dbuf.at[pl.ds(0, nrows)])
            pend_g = [None, None]
            pend_s = [None, None]
            for k in range(nrows):
                b = k & 1
                if pend_s[b] is not None:
                    pend_s[b].wait()
                    pend_s[b] = None
                pend_g[b] = pltpu.async_copy(
                    u_hbm.at[sbuf.at[k]], rbufs[b], gsems[b])
                if k:
                    pb = (k - 1) & 1
                    pend_g[pb].wait()
                    pend_s[pb] = pltpu.async_copy(
                        rbufs[pb], acc.at[dbuf.at[k - 1]], ssems[pb],
                        add=True)
            lb = (nrows - 1) & 1
            pend_g[lb].wait()
            pltpu.async_copy(rbufs[lb], acc.at[dbuf.at[nrows - 1]],
                             ssems[lb], add=True).wait()
            if pend_s[1 - lb] is not None:
                pend_s[1 - lb].wait()

        base = w * RPW

        def outer(j, carry):
            do_rows(base + j * CHUNK, CHUNK)
            return carry

        lax.fori_loop(0, NFULL, outer, 0)
        do_rows(base + NFULL * CHUNK, NTAIL)

        @pl.when(w < REM)
        def _():
            do_rows(NW * RPW + w, 1)

        plsc.subcore_barrier()
        _acc_writeback(acc, out_hbm, c, s)

    return pl.kernel(
        body,
        out_type=jax.ShapeDtypeStruct((NCORES,) + tshape, jnp.float32),
        mesh=_mesh(),
        compiler_params=pltpu.CompilerParams(use_tc_tiling_on_sc=False, needs_layout_passes=False),
        scratch_types=[
            pltpu.VMEM((CHUNK, IDXW), jnp.int32),
            pltpu.VMEM((CHUNK, IDXW), jnp.int32),
            pltpu.VMEM(rshape, jnp.float32),
            pltpu.VMEM(rshape, jnp.float32),
            pltpu.VMEM_SHARED(tshape, jnp.float32),
            pltpu.SemaphoreType.DMA,
            pltpu.SemaphoreType.DMA,
            pltpu.SemaphoreType.DMA,
            pltpu.SemaphoreType.DMA,
        ],
    )


_EDGE_AGG_CACHE = {}


def _edge_agg(D):
    if D not in _EDGE_AGG_CACHE:
        _EDGE_AGG_CACHE[D] = _make_edge_agg(D)
    return _EDGE_AGG_CACHE[D]


def _agg8(u, src2d, dst2d, zeros):
    return _edge_agg(8)(u, src2d, dst2d, zeros)


def _agg32(u, src2d, dst2d, zeros):
    return _edge_agg(32)(u, src2d, dst2d, zeros)


def _scalar_agg_body(u_hbm, src_hbm, dst_hbm, zero_hbm, out_hbm,
                     sbuf, dbuf, gtab, ub0, ub1, acc, ss0, ss1):
    c = lax.axis_index("c")
    s = lax.axis_index("s")
    w = c * NSUB + s
    _acc_zero(zero_hbm, acc, s)
    pltpu.sync_copy(u_hbm, gtab)
    plsc.subcore_barrier()
    ubufs = (ub0, ub1)
    ssems = (ss0, ss1)

    def do_rows(row0, nrows):
        pltpu.sync_copy(src_hbm.at[pl.ds(row0, nrows)],
                        sbuf.at[pl.ds(0, nrows)])
        pltpu.sync_copy(dst_hbm.at[pl.ds(row0, nrows)],
                        dbuf.at[pl.ds(0, nrows)])
        pend_s = [None, None]
        for k in range(nrows):
            b = k & 1
            if pend_s[b] is not None:
                pend_s[b].wait()
                pend_s[b] = None
            for l in range(IDXW // 16):
                idx = sbuf[k, pl.ds(l * 16, 16)]
                ubufs[b][pl.ds(l * 16, 16)] = plsc.load_gather(gtab, [idx])
            pend_s[b] = pltpu.async_copy(
                ubufs[b], acc.at[dbuf.at[k]], ssems[b], add=True)
        for b in (0, 1):
            if pend_s[b] is not None:
                pend_s[b].wait()

    base = w * RPW

    def outer(j, carry):
        do_rows(base + j * CHUNK, CHUNK)
        return carry

    lax.fori_loop(0, NFULL, outer, 0)
    do_rows(base + NFULL * CHUNK, NTAIL)

    @pl.when(w < REM)
    def _():
        do_rows(NW * RPW + w, 1)

    plsc.subcore_barrier()
    _acc_writeback(acc, out_hbm, c, s)


def _agg_scalar(u, src2d, dst2d, zeros):
    if "scalar" not in _EDGE_AGG_CACHE:
        _EDGE_AGG_CACHE["scalar"] = pl.kernel(
            _scalar_agg_body,
            out_type=jax.ShapeDtypeStruct((NCORES, N), jnp.float32),
            mesh=_mesh(),
            compiler_params=pltpu.CompilerParams(use_tc_tiling_on_sc=False, needs_layout_passes=False),
            scratch_types=[
                pltpu.VMEM((CHUNK, IDXW), jnp.int32),
                pltpu.VMEM((CHUNK, IDXW), jnp.int32),
                pltpu.VMEM((N,), jnp.float32),
                pltpu.VMEM((IDXW,), jnp.float32),
                pltpu.VMEM((IDXW,), jnp.float32),
                pltpu.VMEM_SHARED((N,), jnp.float32),
                pltpu.SemaphoreType.DMA,
                pltpu.SemaphoreType.DMA,
            ],
        )
    return _EDGE_AGG_CACHE["scalar"](u, src2d, dst2d, zeros)


def _edge_count_body(dst_hbm, zero_hbm, out_hbm, dbuf, ones, acc):
    c = lax.axis_index("c")
    s = lax.axis_index("s")
    w = c * NSUB + s
    _acc_zero(zero_hbm, acc, s)

    def fill(k, carry):
        ones[pl.ds(k * 16, 16)] = jnp.ones((16,), jnp.float32)
        return carry

    lax.fori_loop(0, IDXW // 16, fill, 0)
    plsc.subcore_barrier()

    def do_rows(row0, nrows):
        pltpu.sync_copy(dst_hbm.at[pl.ds(row0, nrows)],
                        dbuf.at[pl.ds(0, nrows)])

        def inner(k, carry):
            pltpu.sync_copy(ones, acc.at[dbuf.at[k]], add=True)
            return carry

        lax.fori_loop(0, nrows, inner, 0)

    base = w * RPW

    def outer(j, carry):
        do_rows(base + j * CHUNK, CHUNK)
        return carry

    lax.fori_loop(0, NFULL, outer, 0)
    do_rows(base + NFULL * CHUNK, NTAIL)

    @pl.when(w < REM)
    def _():
        do_rows(NW * RPW + w, 1)

    plsc.subcore_barrier()
    _acc_writeback(acc, out_hbm, c, s)


def _edge_count(dst2d, zeros):
    if "count" not in _EDGE_AGG_CACHE:
        _EDGE_AGG_CACHE["count"] = pl.kernel(
            _edge_count_body,
            out_type=jax.ShapeDtypeStruct((NCORES, N), jnp.float32),
            mesh=_mesh(),
            compiler_params=pltpu.CompilerParams(use_tc_tiling_on_sc=False, needs_layout_passes=False),
            scratch_types=[
                pltpu.VMEM((CHUNK, IDXW), jnp.int32),
                pltpu.VMEM((IDXW,), jnp.float32),
                pltpu.VMEM_SHARED((N,), jnp.float32),
            ],
        )
    return _EDGE_AGG_CACHE["count"](dst2d, zeros)


BLK = 1024
GRID = NPAD // BLK


def _tc_a_body(deg_ref, x_ref, dinv_ref, u1_ref):
    deg = deg_ref[0] + deg_ref[1] + 1.0
    dinv = lax.rsqrt(deg)
    dinv_ref[...] = dinv
    u1_ref[...] = dinv * x_ref[...]


def _tc_a(deg_planes, xpad):
    return pl.pallas_call(
        _tc_a_body,
        grid=(GRID,),
        in_specs=[
            pl.BlockSpec((2, BLK, 1), lambda i: (0, i, 0)),
            pl.BlockSpec((BLK, 8), lambda i: (i, 0)),
        ],
        out_specs=[
            pl.BlockSpec((BLK, 1), lambda i: (i, 0)),
            pl.BlockSpec((BLK, 8), lambda i: (i, 0)),
        ],
        out_shape=[
            jax.ShapeDtypeStruct((NPAD, 1), jnp.float32),
            jax.ShapeDtypeStruct((NPAD, 8), jnp.float32),
        ],
    )(deg_planes, xpad)


def _tc_b_body(agg_ref, x_ref, dinv_ref, W1_ref, b1_ref, Ws_ref,
               h1_ref, s_ref, su_ref):
    dinv = dinv_ref[...]
    agg = agg_ref[0] + agg_ref[1]
    z = dinv * agg + (dinv * dinv) * x_ref[...]
    h1 = jnp.maximum(jnp.dot(z, W1_ref[...],
                             preferred_element_type=jnp.float32)
                     + b1_ref[...], 0.0)
    h1_ref[...] = h1
    s = jnp.dot(h1, Ws_ref[...], preferred_element_type=jnp.float32)
    s_ref[...] = s
    su_ref[...] = dinv * s


def _tc_b(agg_planes, xpad, dinv, W1p, b1, Ws):
    return pl.pallas_call(
        _tc_b_body,
        grid=(GRID,),
        in_specs=[
            pl.BlockSpec((2, BLK, 8), lambda i: (0, i, 0)),
            pl.BlockSpec((BLK, 8), lambda i: (i, 0)),
            pl.BlockSpec((BLK, 1), lambda i: (i, 0)),
            pl.BlockSpec((8, 64), lambda i: (0, 0)),
            pl.BlockSpec((1, 64), lambda i: (0, 0)),
            pl.BlockSpec((64, 1), lambda i: (0, 0)),
        ],
        out_specs=[
            pl.BlockSpec((BLK, 64), lambda i: (i, 0)),
            pl.BlockSpec((BLK, 1), lambda i: (i, 0)),
            pl.BlockSpec((BLK, 1), lambda i: (i, 0)),
        ],
        out_shape=[
            jax.ShapeDtypeStruct((NPAD, 64), jnp.float32),
            jax.ShapeDtypeStruct((NPAD, 1), jnp.float32),
            jax.ShapeDtypeStruct((NPAD, 1), jnp.float32),
        ],
    )(agg_planes, xpad, dinv, W1p, b1, Ws)


def _tc_select_body(sagg_ref, s_ref, dinv_ref, bs_ref, mask_ref, gate_ref):
    dinv = dinv_ref[...]
    score = dinv * (sagg_ref[0] + sagg_ref[1]) + dinv * dinv * s_ref[...] \
        + bs_ref[0, 0]
    pos = (lax.broadcasted_iota(jnp.int32, (NROW, 128), 0) * 128
           + lax.broadcasted_iota(jnp.int32, (NROW, 128), 1))
    valid = pos < N
    bits = lax.bitcast_convert_type(score, jnp.uint32)
    key = jnp.where(bits >= jnp.uint32(0x80000000), ~bits,
                    bits | jnp.uint32(0x80000000))
    key = jnp.where(valid, key, jnp.uint32(0))

    def bit_step(i, t):
        b = jnp.uint32(31) - i.astype(jnp.uint32)
        cand = t | (jnp.uint32(1) << b)
        cnt = jnp.sum((key >= cand).astype(jnp.float32))
        return jnp.where(cnt >= K, cand, t)

    T = lax.fori_loop(0, 32, bit_step, jnp.uint32(0))
    n_gt = jnp.sum((key > T).astype(jnp.float32))
    need = K - n_gt
    eq = key == T

    def pos_step(i, xacc):
        b = 16 - i
        cand = xacc | (1 << b)
        cnt = jnp.sum(jnp.where(eq & (pos < cand), 1.0, 0.0))
        return jnp.where(cnt < need, cand, xacc)

    X = lax.fori_loop(0, 17, pos_step, 0)
    keep = (key > T) | (eq & (pos <= X) & (need >= 1.0))
    maskf = jnp.where(keep, 1.0, 0.0)
    mask_ref[...] = maskf
    gate_ref[...] = jnp.tanh(score) * maskf


def _tc_select(sagg_planes, s2d, dinv2d, bs):
    return pl.pallas_call(
        _tc_select_body,
        out_shape=[
            jax.ShapeDtypeStruct((NROW, 128), jnp.float32),
            jax.ShapeDtypeStruct((NROW, 128), jnp.float32),
        ],
    )(sagg_planes, s2d, dinv2d, bs)


def _tc_c2_body(h1_ref, gate_ref, W2_ref, hw2_ref):
    hf = h1_ref[...] * gate_ref[...]
    hw2_ref[...] = jnp.dot(hf, W2_ref[...],
                           preferred_element_type=jnp.float32)


def _tc_c2(h1, gate, W2):
    return pl.pallas_call(
        _tc_c2_body,
        grid=(GRID,),
        in_specs=[
            pl.BlockSpec((BLK, 64), lambda i: (i, 0)),
            pl.BlockSpec((BLK, 1), lambda i: (i, 0)),
            pl.BlockSpec((64, 32), lambda i: (0, 0)),
        ],
        out_specs=pl.BlockSpec((BLK, 32), lambda i: (i, 0)),
        out_shape=jax.ShapeDtypeStruct((NPAD, 32), jnp.float32),
    )(h1, gate, W2)


def _tc_d_body(magg_ref, mask_ref, hw2_ref, dinv2_ref, u2_ref):
    m = mask_ref[...]
    deg2 = m * (magg_ref[0] + magg_ref[1] + 1.0)
    dinv2 = jnp.where(deg2 > 0, 1.0 / jnp.sqrt(jnp.maximum(deg2, 1e-12)),
                      0.0)
    dinv2_ref[...] = dinv2
    u2_ref[...] = dinv2 * hw2_ref[...]


def _tc_d(magg_planes, mask, hw2):
    return pl.pallas_call(
        _tc_d_body,
        grid=(GRID,),
        in_specs=[
            pl.BlockSpec((2, BLK, 1), lambda i: (0, i, 0)),
            pl.BlockSpec((BLK, 1), lambda i: (i, 0)),
            pl.BlockSpec((BLK, 32), lambda i: (i, 0)),
        ],
        out_specs=[
            pl.BlockSpec((BLK, 1), lambda i: (i, 0)),
            pl.BlockSpec((BLK, 32), lambda i: (i, 0)),
        ],
        out_shape=[
            jax.ShapeDtypeStruct((NPAD, 1), jnp.float32),
            jax.ShapeDtypeStruct((NPAD, 32), jnp.float32),
        ],
    )(magg_planes, mask, hw2)


def _tc_e_body(agg2_ref, hw2_ref, dinv2_ref, mask_ref, b2_ref, t32_ref):
    i = pl.program_id(0)

    @pl.when(i == 0)
    def _():
        t32_ref[...] = jnp.zeros((1, 32), jnp.float32)

    dinv2 = dinv2_ref[...]
    agg2 = agg2_ref[0] + agg2_ref[1]
    h2 = jnp.maximum(dinv2 * agg2 + (dinv2 * dinv2) * hw2_ref[...]
                     + b2_ref[...], 0.0)
    t32_ref[...] += jnp.sum(h2 * mask_ref[...], axis=0, keepdims=True)


def _tc_e():
    return pl.pallas_call(
        _tc_e_body,
        grid=(GRID,),
        in_specs=[
            pl.BlockSpec((2, BLK, 32), lambda i: (0, i, 0)),
            pl.BlockSpec((BLK, 32), lambda i: (i, 0)),
            pl.BlockSpec((BLK, 1), lambda i: (i, 0)),
            pl.BlockSpec((BLK, 1), lambda i: (i, 0)),
            pl.BlockSpec((1, 32), lambda i: (0, 0)),
        ],
        out_specs=pl.BlockSpec((1, 32), lambda i: (0, 0)),
        out_shape=jax.ShapeDtypeStruct((1, 32), jnp.float32),
    )


def _tc_final_body(t32_ref, fc1W_ref, fc1b_ref, c_ref, o_ref):
    t = t32_ref[0, :]
    w = fc1W_ref[0, :]
    csum = jnp.sum(c_ref[0, :])
    m = (jnp.sum(t * w) + K * fc1b_ref[0, 0]) * csum / (4.0 * K)
    o_ref[...] = jnp.full((1, 1), m, jnp.float32)


def _padn(v):
    return jnp.pad(v, (0, NPAD - N))[:, None]


def kernel(x, edge_index, batch, y, W1, b1, Ws, bs, W2, b2, fc1_W, fc1_b,
           fc2_W, fc2_b, fc3_W, fc3_b, fc4_W, fc4_b, fc5_W, fc5_b,
           ct_vec, cs_vec, al_vec, cet_vec):
    src2d = edge_index[0].reshape(ROWS, IDXW)
    dst2d = edge_index[1].reshape(ROWS, IDXW)
    zeros1 = jnp.zeros((N,), jnp.float32)
    zeros8 = jnp.zeros((N, 8), jnp.float32)
    zeros32 = jnp.zeros((N, 32), jnp.float32)
    xpad = jnp.pad(x, ((0, NPAD - N), (0, 2)))
    W1p = jnp.pad(W1, ((0, 2), (0, 0)))

    deg_pl = _edge_count(dst2d, zeros1)
    deg_planes = jnp.pad(deg_pl, ((0, 0), (0, NPAD - N)))[:, :, None]
    dinv, u1 = _tc_a(deg_planes, xpad)

    agg1_pl = _agg8(u1[:N], src2d, dst2d, zeros8)
    agg1_planes = jnp.pad(agg1_pl, ((0, 0), (0, NPAD - N), (0, 0)))
    h1, s, su = _tc_b(agg1_planes, xpad, dinv, W1p, b1[None, :], Ws)

    sagg_pl = _agg_scalar(su[:N, 0], src2d, dst2d, zeros1)
    sagg2d = jnp.pad(sagg_pl, ((0, 0), (0, NPAD - N))).reshape(2, NROW, 128)
    mask2d, gate2d = _tc_select(sagg2d, s.reshape(NROW, 128),
                                dinv.reshape(NROW, 128), bs[None, :])
    mask = mask2d.reshape(NPAD, 1)
    gate = gate2d.reshape(NPAD, 1)

    hw2 = _tc_c2(h1, gate, W2)
    magg_pl = _agg_scalar(mask[:N, 0], src2d, dst2d, zeros1)
    magg_planes = jnp.pad(magg_pl, ((0, 0), (0, NPAD - N)))[:, :, None]
    dinv2, u2 = _tc_d(magg_planes, mask, hw2)
    agg2_pl = _agg32(u2[:N], src2d, dst2d, zeros32)
    agg2_planes = jnp.pad(agg2_pl, ((0, 0), (0, NPAD - N), (0, 0)))
    t32 = _tc_e()(agg2_planes, hw2, dinv2, mask, b2[None, :])

    c = jnp.concatenate([
        fc2_W @ ct_vec + fc2_b, fc3_W @ cs_vec + fc3_b,
        fc4_W @ al_vec + fc4_b, fc5_W @ cet_vec + fc5_b])
    out = pl.pallas_call(
        _tc_final_body,
        out_shape=jax.ShapeDtypeStruct((1, 1), jnp.float32),
    )(t32, fc1_W, fc1_b[None, :], c[None, :])
    return out[0]


# speedup vs baseline: 106.8140x; 1.1351x over previous
import jax
import jax.numpy as jnp
from jax import lax
from jax.experimental import pallas as pl
from jax.experimental.pallas import tpu as pltpu
from jax.experimental.pallas import tpu_sc as plsc

N = 50000
E = 1600000
K = 35000

NCORES = 2
NSUB = 16
NW = NCORES * NSUB
IDXW = 128
ROWS = E // IDXW
RPW = ROWS // NW
REM = ROWS - RPW * NW
CHUNK = 32
NFULL = RPW // CHUNK
NTAIL = RPW % CHUNK
NBUF = 4
ZR = 3200
ZR_LAST = N - 15 * ZR

NPAD = 50176
NROW = NPAD // 128

BLK = 1000
GRID = N // BLK


def _mesh():
    return plsc.VectorSubcoreMesh(core_axis_name="c", subcore_axis_name="s")


def _cparams():
    return pltpu.CompilerParams(use_tc_tiling_on_sc=False,
                                needs_layout_passes=False)


def _acc_zero(zero_hbm, acc, s):
    @pl.when(s < 15)
    def _():
        pltpu.sync_copy(zero_hbm.at[pl.ds(s * ZR, ZR)],
                        acc.at[pl.ds(s * ZR, ZR)])

    @pl.when(s == 15)
    def _():
        pltpu.sync_copy(zero_hbm.at[pl.ds(15 * ZR, ZR_LAST)],
                        acc.at[pl.ds(15 * ZR, ZR_LAST)])


def _acc_writeback(acc, out_hbm, c, s):
    @pl.when(s < 15)
    def _():
        pltpu.sync_copy(acc.at[pl.ds(s * ZR, ZR)],
                        out_hbm.at[c].at[pl.ds(s * ZR, ZR)])

    @pl.when(s == 15)
    def _():
        pltpu.sync_copy(acc.at[pl.ds(15 * ZR, ZR_LAST)],
                        out_hbm.at[c].at[pl.ds(15 * ZR, ZR_LAST)])


def _edge_sweep(w, do_rows):
    base = w * RPW

    def outer(j, carry):
        do_rows(base + j * CHUNK, CHUNK)
        return carry

    lax.fori_loop(0, NFULL, outer, 0)
    do_rows(base + NFULL * CHUNK, NTAIL)

    @pl.when(w < REM)
    def _():
        do_rows(NW * RPW + w, 1)


def _make_edge_agg(D):
    def body(u_hbm, src_hbm, dst_hbm, zero_hbm, out_hbm,
             sbuf, dbuf, rbufs, acc, gsems, ssems):
        c = lax.axis_index("c")
        s = lax.axis_index("s")
        w = c * NSUB + s
        _acc_zero(zero_hbm, acc, s)
        plsc.subcore_barrier()

        def do_rows(row0, nrows):
            pltpu.sync_copy(src_hbm.at[pl.ds(row0, nrows)],
                            sbuf.at[pl.ds(0, nrows)])
            pltpu.sync_copy(dst_hbm.at[pl.ds(row0, nrows)],
                            dbuf.at[pl.ds(0, nrows)])
            pend_g = [None] * NBUF
            pend_s = [None] * NBUF
            for k in range(nrows):
                b = k % NBUF
                if pend_s[b] is not None:
                    pend_s[b].wait()
                    pend_s[b] = None
                pend_g[b] = pltpu.async_copy(
                    u_hbm.at[sbuf.at[k]], rbufs[b], gsems[b])
                if k:
                    pb = (k - 1) % NBUF
                    pend_g[pb].wait()
                    pend_s[pb] = pltpu.async_copy(
                        rbufs[pb], acc.at[dbuf.at[k - 1]], ssems[pb],
                        add=True)
            lb = (nrows - 1) % NBUF
            pend_g[lb].wait()
            pend_s[lb] = pltpu.async_copy(
                rbufs[lb], acc.at[dbuf.at[nrows - 1]], ssems[lb], add=True)
            for b in range(NBUF):
                if pend_s[b] is not None:
                    pend_s[b].wait()

        _edge_sweep(w, do_rows)
        plsc.subcore_barrier()
        _acc_writeback(acc, out_hbm, c, s)

    return pl.kernel(
        body,
        out_type=jax.ShapeDtypeStruct((NCORES, N, D), jnp.float32),
        mesh=_mesh(),
        compiler_params=_cparams(),
        scratch_types=[
            pltpu.VMEM((CHUNK, IDXW), jnp.int32),
            pltpu.VMEM((CHUNK, IDXW), jnp.int32),
            tuple(pltpu.VMEM((IDXW, D), jnp.float32) for _ in range(NBUF)),
            pltpu.VMEM_SHARED((N, D), jnp.float32),
            tuple(pltpu.SemaphoreType.DMA for _ in range(NBUF)),
            tuple(pltpu.SemaphoreType.DMA for _ in range(NBUF)),
        ],
    )


_EDGE_AGG_CACHE = {}


def _edge_agg(D):
    if D not in _EDGE_AGG_CACHE:
        _EDGE_AGG_CACHE[D] = _make_edge_agg(D)
    return _EDGE_AGG_CACHE[D]


def _agg8(u, src2d, dst2d, zeros):
    return _edge_agg(8)(u, src2d, dst2d, zeros)


def _agg32(u, src2d, dst2d, zeros):
    return _edge_agg(32)(u, src2d, dst2d, zeros)


def _scalar_agg_body(u_hbm, src_hbm, dst_hbm, zero_hbm, out_hbm,
                     sbuf, dbuf, gtab, ubufs, acc, ssems):
    c = lax.axis_index("c")
    s = lax.axis_index("s")
    w = c * NSUB + s
    _acc_zero(zero_hbm, acc, s)
    pltpu.sync_copy(u_hbm, gtab)
    plsc.subcore_barrier()

    def do_rows(row0, nrows):
        pltpu.sync_copy(src_hbm.at[pl.ds(row0, nrows)],
                        sbuf.at[pl.ds(0, nrows)])
        pltpu.sync_copy(dst_hbm.at[pl.ds(row0, nrows)],
                        dbuf.at[pl.ds(0, nrows)])
        pend_s = [None, None]
        for k in range(nrows):
            b = k & 1
            if pend_s[b] is not None:
                pend_s[b].wait()
                pend_s[b] = None
            for l in range(IDXW // 16):
                idx = sbuf[k, pl.ds(l * 16, 16)]
                ubufs[b][pl.ds(l * 16, 16)] = plsc.load_gather(gtab, [idx])
            pend_s[b] = pltpu.async_copy(
                ubufs[b], acc.at[dbuf.at[k]], ssems[b], add=True)
        for b in (0, 1):
            if pend_s[b] is not None:
                pend_s[b].wait()

    _edge_sweep(w, do_rows)
    plsc.subcore_barrier()
    _acc_writeback(acc, out_hbm, c, s)


def _agg_scalar(u, src2d, dst2d, zeros):
    if "scalar" not in _EDGE_AGG_CACHE:
        _EDGE_AGG_CACHE["scalar"] = pl.kernel(
            _scalar_agg_body,
            out_type=jax.ShapeDtypeStruct((NCORES, N), jnp.float32),
            mesh=_mesh(),
            compiler_params=_cparams(),
            scratch_types=[
                pltpu.VMEM((CHUNK, IDXW), jnp.int32),
                pltpu.VMEM((CHUNK, IDXW), jnp.int32),
                pltpu.VMEM((N,), jnp.float32),
                (pltpu.VMEM((IDXW,), jnp.float32),
                 pltpu.VMEM((IDXW,), jnp.float32)),
                pltpu.VMEM_SHARED((N,), jnp.float32),
                (pltpu.SemaphoreType.DMA, pltpu.SemaphoreType.DMA),
            ],
        )
    return _EDGE_AGG_CACHE["scalar"](u, src2d, dst2d, zeros)


def _edge_count_body(dst_hbm, zero_hbm, out_hbm, dbuf, ones, acc):
    c = lax.axis_index("c")
    s = lax.axis_index("s")
    w = c * NSUB + s
    _acc_zero(zero_hbm, acc, s)

    def fill(k, carry):
        ones[pl.ds(k * 16, 16)] = jnp.ones((16,), jnp.float32)
        return carry

    lax.fori_loop(0, IDXW // 16, fill, 0)
    plsc.subcore_barrier()

    def do_rows(row0, nrows):
        pltpu.sync_copy(dst_hbm.at[pl.ds(row0, nrows)],
                        dbuf.at[pl.ds(0, nrows)])

        def inner(k, carry):
            pltpu.sync_copy(ones, acc.at[dbuf.at[k]], add=True)
            return carry

        lax.fori_loop(0, nrows, inner, 0)

    _edge_sweep(w, do_rows)
    plsc.subcore_barrier()
    _acc_writeback(acc, out_hbm, c, s)


def _edge_count(dst2d, zeros):
    if "count" not in _EDGE_AGG_CACHE:
        _EDGE_AGG_CACHE["count"] = pl.kernel(
            _edge_count_body,
            out_type=jax.ShapeDtypeStruct((NCORES, N), jnp.float32),
            mesh=_mesh(),
            compiler_params=_cparams(),
            scratch_types=[
                pltpu.VMEM((CHUNK, IDXW), jnp.int32),
                pltpu.VMEM((IDXW,), jnp.float32),
                pltpu.VMEM_SHARED((N,), jnp.float32),
            ],
        )
    return _EDGE_AGG_CACHE["count"](dst2d, zeros)


def _tc_a_body(deg_ref, x_ref, dinv_ref, u1_ref):
    deg = deg_ref[0] + deg_ref[1] + 1.0
    dinv = lax.rsqrt(deg)
    dinv_ref[...] = dinv
    u1_ref[...] = dinv * x_ref[...]


def _tc_a(deg_planes, x8):
    return pl.pallas_call(
        _tc_a_body,
        grid=(GRID,),
        in_specs=[
            pl.BlockSpec((2, BLK, 1), lambda i: (0, i, 0)),
            pl.BlockSpec((BLK, 8), lambda i: (i, 0)),
        ],
        out_specs=[
            pl.BlockSpec((BLK, 1), lambda i: (i, 0)),
            pl.BlockSpec((BLK, 8), lambda i: (i, 0)),
        ],
        out_shape=[
            jax.ShapeDtypeStruct((N, 1), jnp.float32),
            jax.ShapeDtypeStruct((N, 8), jnp.float32),
        ],
    )(deg_planes, x8)


def _tc_b_body(agg_ref, x_ref, dinv_ref, W1_ref, b1_ref, Ws_ref,
               h1_ref, s_ref, su_ref):
    dinv = dinv_ref[...]
    agg = agg_ref[0] + agg_ref[1]
    z = dinv * agg + (dinv * dinv) * x_ref[...]
    h1 = jnp.maximum(jnp.dot(z, W1_ref[...],
                             preferred_element_type=jnp.float32)
                     + b1_ref[...], 0.0)
    h1_ref[...] = h1
    s = jnp.dot(h1, Ws_ref[...], preferred_element_type=jnp.float32)
    s_ref[...] = s
    su_ref[...] = dinv * s


def _tc_b(agg_planes, x8, dinv, W1p, b1, Ws):
    return pl.pallas_call(
        _tc_b_body,
        grid=(GRID,),
        in_specs=[
            pl.BlockSpec((2, BLK, 8), lambda i: (0, i, 0)),
            pl.BlockSpec((BLK, 8), lambda i: (i, 0)),
            pl.BlockSpec((BLK, 1), lambda i: (i, 0)),
            pl.BlockSpec((8, 64), lambda i: (0, 0)),
            pl.BlockSpec((1, 64), lambda i: (0, 0)),
            pl.BlockSpec((64, 1), lambda i: (0, 0)),
        ],
        out_specs=[
            pl.BlockSpec((BLK, 64), lambda i: (i, 0)),
            pl.BlockSpec((BLK, 1), lambda i: (i, 0)),
            pl.BlockSpec((BLK, 1), lambda i: (i, 0)),
        ],
        out_shape=[
            jax.ShapeDtypeStruct((N, 64), jnp.float32),
            jax.ShapeDtypeStruct((N, 1), jnp.float32),
            jax.ShapeDtypeStruct((N, 1), jnp.float32),
        ],
    )(agg_planes, x8, dinv, W1p, b1, Ws)


def _tc_select_body(sagg_ref, s_ref, dinv_ref, bs_ref, mask_ref, gate_ref):
    dinv = dinv_ref[...]
    score = dinv * (sagg_ref[0] + sagg_ref[1]) + dinv * dinv * s_ref[...] \
        + bs_ref[0, 0]
    pos = (lax.broadcasted_iota(jnp.int32, (NROW, 128), 0) * 128
           + lax.broadcasted_iota(jnp.int32, (NROW, 128), 1))
    valid = pos < N
    bits = lax.bitcast_convert_type(score, jnp.uint32)
    key = jnp.where(bits >= jnp.uint32(0x80000000), ~bits,
                    bits | jnp.uint32(0x80000000))
    key = jnp.where(valid, key, jnp.uint32(0))

    def bit_step(i, t):
        b = jnp.uint32(31) - i.astype(jnp.uint32)
        cand = t | (jnp.uint32(1) << b)
        cnt = jnp.sum((key >= cand).astype(jnp.float32))
        return jnp.where(cnt >= K, cand, t)

    T = lax.fori_loop(0, 32, bit_step, jnp.uint32(0))
    n_gt = jnp.sum((key > T).astype(jnp.float32))
    need = K - n_gt
    eq = key == T

    def pos_step(i, xacc):
        b = 16 - i
        cand = xacc | (1 << b)
        cnt = jnp.sum(jnp.where(eq & (pos < cand), 1.0, 0.0))
        return jnp.where(cnt < need, cand, xacc)

    X = lax.fori_loop(0, 17, pos_step, 0)
    keep = (key > T) | (eq & (pos <= X) & (need >= 1.0))
    maskf = jnp.where(keep, 1.0, 0.0)
    mask_ref[...] = maskf
    gate_ref[...] = jnp.tanh(score) * maskf


def _tc_select(sagg_planes, s2d, dinv2d, bs):
    return pl.pallas_call(
        _tc_select_body,
        out_shape=[
            jax.ShapeDtypeStruct((NROW, 128), jnp.float32),
            jax.ShapeDtypeStruct((NROW, 128), jnp.float32),
        ],
    )(sagg_planes, s2d, dinv2d, bs)


def _tc_d_body(magg_ref, mask_ref, gate_ref, h1_ref, W2_ref,
               hw2_ref, dinv2_ref, u2_ref):
    hf = h1_ref[...] * gate_ref[...]
    hw2 = jnp.dot(hf, W2_ref[...], preferred_element_type=jnp.float32)
    hw2_ref[...] = hw2
    m = mask_ref[...]
    deg2 = m * (magg_ref[0] + magg_ref[1] + 1.0)
    dinv2 = jnp.where(deg2 > 0, 1.0 / jnp.sqrt(jnp.maximum(deg2, 1e-12)),
                      0.0)
    dinv2_ref[...] = dinv2
    u2_ref[...] = dinv2 * hw2


def _tc_d(magg_planes, mask, gate, h1, W2):
    return pl.pallas_call(
        _tc_d_body,
        grid=(GRID,),
        in_specs=[
            pl.BlockSpec((2, BLK, 1), lambda i: (0, i, 0)),
            pl.BlockSpec((BLK, 1), lambda i: (i, 0)),
            pl.BlockSpec((BLK, 1), lambda i: (i, 0)),
            pl.BlockSpec((BLK, 64), lambda i: (i, 0)),
            pl.BlockSpec((64, 32), lambda i: (0, 0)),
        ],
        out_specs=[
            pl.BlockSpec((BLK, 32), lambda i: (i, 0)),
            pl.BlockSpec((BLK, 1), lambda i: (i, 0)),
            pl.BlockSpec((BLK, 32), lambda i: (i, 0)),
        ],
        out_shape=[
            jax.ShapeDtypeStruct((N, 32), jnp.float32),
            jax.ShapeDtypeStruct((N, 1), jnp.float32),
            jax.ShapeDtypeStruct((N, 32), jnp.float32),
        ],
    )(magg_planes, mask, gate, h1, W2)


def _tc_ef_body(agg2_ref, hw2_ref, dinv2_ref, mask_ref, b2_ref,
                fc1W_ref, fc1b_ref, fc2W_ref, ct_ref, fc3W_ref, cs_ref,
                fc4W_ref, al_ref, fc5W_ref, cet_ref, fc2b_ref, fc3b_ref,
                fc4b_ref, fc5b_ref, o_ref, t32_acc):
    i = pl.program_id(0)

    @pl.when(i == 0)
    def _():
        t32_acc[...] = jnp.zeros((1, 32), jnp.float32)

    dinv2 = dinv2_ref[...]
    agg2 = agg2_ref[0] + agg2_ref[1]
    h2 = jnp.maximum(dinv2 * agg2 + (dinv2 * dinv2) * hw2_ref[...]
                     + b2_ref[...], 0.0)
    t32_acc[...] += jnp.sum(h2 * mask_ref[...], axis=0, keepdims=True)

    @pl.when(i == GRID - 1)
    def _():
        csum = (jnp.sum(fc2W_ref[...] * ct_ref[...]) + fc2b_ref[0, 0]
                + jnp.sum(fc3W_ref[...] * cs_ref[...]) + fc3b_ref[0, 0]
                + jnp.sum(fc4W_ref[...] * al_ref[...]) + fc4b_ref[0, 0]
                + jnp.sum(fc5W_ref[...] * cet_ref[...]) + fc5b_ref[0, 0])
        tot = jnp.sum(t32_acc[...] * fc1W_ref[...]) + K * fc1b_ref[0, 0]
        o_ref[...] = jnp.full((1, 1), tot * csum / (4.0 * K), jnp.float32)


def _tc_ef(agg2_planes, hw2, dinv2, mask, b2, fc1_W, fc1_b,
           fc2_W, ct_vec, fc3_W, cs_vec, fc4_W, al_vec, fc5_W, cet_vec,
           fc2_b, fc3_b, fc4_b, fc5_b):
    const = lambda i: (0, 0)
    return pl.pallas_call(
        _tc_ef_body,
        grid=(GRID,),
        in_specs=[
            pl.BlockSpec((2, BLK, 32), lambda i: (0, i, 0)),
            pl.BlockSpec((BLK, 32), lambda i: (i, 0)),
            pl.BlockSpec((BLK, 1), lambda i: (i, 0)),
            pl.BlockSpec((BLK, 1), lambda i: (i, 0)),
            pl.BlockSpec((1, 32), const),
            pl.BlockSpec((1, 32), const),
            pl.BlockSpec((1, 1), const),
            pl.BlockSpec((1, 33), const),
            pl.BlockSpec((1, 33), const),
            pl.BlockSpec((1, 25), const),
            pl.BlockSpec((1, 25), const),
            pl.BlockSpec((1, 52), const),
            pl.BlockSpec((1, 52), const),
            pl.BlockSpec((1, 10), const),
            pl.BlockSpec((1, 10), const),
            pl.BlockSpec((1, 1), const),
            pl.BlockSpec((1, 1), const),
            pl.BlockSpec((1, 1), const),
            pl.BlockSpec((1, 1), const),
        ],
        out_specs=pl.BlockSpec((1, 1), const),
        out_shape=jax.ShapeDtypeStruct((1, 1), jnp.float32),
        scratch_shapes=[pltpu.VMEM((1, 32), jnp.float32)],
    )(agg2_planes, hw2, dinv2, mask, b2, fc1_W, fc1_b,
      fc2_W, ct_vec, fc3_W, cs_vec, fc4_W, al_vec, fc5_W, cet_vec,
      fc2_b, fc3_b, fc4_b, fc5_b)


def _pad2d(v):
    return jnp.pad(v, (0, NPAD - N)).reshape(NROW, 128)


def kernel(x, edge_index, batch, y, W1, b1, Ws, bs, W2, b2, fc1_W, fc1_b,
           fc2_W, fc2_b, fc3_W, fc3_b, fc4_W, fc4_b, fc5_W, fc5_b,
           ct_vec, cs_vec, al_vec, cet_vec):
    src2d = edge_index[0].reshape(ROWS, IDXW)
    dst2d = edge_index[1].reshape(ROWS, IDXW)
    zeros1 = jnp.zeros((N,), jnp.float32)
    zeros8 = jnp.zeros((N, 8), jnp.float32)
    zeros32 = jnp.zeros((N, 32), jnp.float32)
    x8 = jnp.pad(x, ((0, 0), (0, 2)))
    W1p = jnp.pad(W1, ((0, 2), (0, 0)))

    deg_pl = _edge_count(dst2d, zeros1)
    dinv, u1 = _tc_a(deg_pl[:, :, None], x8)

    agg1_pl = _agg8(u1, src2d, dst2d, zeros8)
    h1, s, su = _tc_b(agg1_pl, x8, dinv, W1p, b1[None, :], Ws)

    sagg_pl = _agg_scalar(su[:, 0], src2d, dst2d, zeros1)
    sagg2d = jnp.pad(sagg_pl, ((0, 0), (0, NPAD - N))).reshape(2, NROW, 128)
    mask2d, gate2d = _tc_select(sagg2d, _pad2d(s[:, 0]), _pad2d(dinv[:, 0]),
                                bs[None, :])
    mask = mask2d.reshape(NPAD)[:N, None]
    gate = gate2d.reshape(NPAD)[:N, None]

    magg_pl = _agg_scalar(mask[:, 0], src2d, dst2d, zeros1)
    hw2, dinv2, u2 = _tc_d(magg_pl[:, :, None], mask, gate, h1, W2)
    agg2_pl = _agg32(u2, src2d, dst2d, zeros32)

    out = _tc_ef(agg2_pl, hw2, dinv2, mask, b2[None, :], fc1_W,
                 fc1_b[None, :], fc2_W, ct_vec[None, :], fc3_W,
                 cs_vec[None, :], fc4_W, al_vec[None, :], fc5_W,
                 cet_vec[None, :], fc2_b[None, :], fc3_b[None, :],
                 fc4_b[None, :], fc5_b[None, :])
    return out[0]


# speedup vs baseline: 118.7466x; 1.1117x over previous
import jax
import jax.numpy as jnp
from jax import lax
from jax.experimental import pallas as pl
from jax.experimental.pallas import tpu as pltpu
from jax.experimental.pallas import tpu_sc as plsc

N = 50000
E = 1600000
K = 35000

NCORES = 2
NSUB = 16
NW = NCORES * NSUB
IDXW = 512
ROWS = E // IDXW
RPW = ROWS // NW
REM = ROWS - RPW * NW
CHUNK = 8
NFULL = RPW // CHUNK
NTAIL = RPW % CHUNK
NBUF = 4
IDXW32 = 256
ROWS32 = E // IDXW32
RPW32 = ROWS32 // NW
REM32 = ROWS32 - RPW32 * NW
CHUNK32 = 8
NFULL32 = RPW32 // CHUNK32
NTAIL32 = RPW32 % CHUNK32
NBUF32 = 2
ZR = 3200
ZR_LAST = N - 15 * ZR

NPAD = 50176
NROW = NPAD // 128

BLK = 1000
GRID = N // BLK


def _mesh():
    return plsc.VectorSubcoreMesh(core_axis_name="c", subcore_axis_name="s")


def _cparams():
    return pltpu.CompilerParams(use_tc_tiling_on_sc=False,
                                needs_layout_passes=False)


def _acc_zero(zero_hbm, acc, s):
    @pl.when(s < 15)
    def _():
        pltpu.sync_copy(zero_hbm.at[pl.ds(s * ZR, ZR)],
                        acc.at[pl.ds(s * ZR, ZR)])

    @pl.when(s == 15)
    def _():
        pltpu.sync_copy(zero_hbm.at[pl.ds(15 * ZR, ZR_LAST)],
                        acc.at[pl.ds(15 * ZR, ZR_LAST)])


def _acc_writeback(acc, out_hbm, c, s):
    @pl.when(s < 15)
    def _():
        pltpu.sync_copy(acc.at[pl.ds(s * ZR, ZR)],
                        out_hbm.at[c].at[pl.ds(s * ZR, ZR)])

    @pl.when(s == 15)
    def _():
        pltpu.sync_copy(acc.at[pl.ds(15 * ZR, ZR_LAST)],
                        out_hbm.at[c].at[pl.ds(15 * ZR, ZR_LAST)])


def _edge_sweep(w, do_rows, rpw=RPW, rem=REM, chunk=CHUNK, nfull=NFULL,
                ntail=NTAIL):
    base = w * rpw

    def outer(j, carry):
        do_rows(base + j * chunk, chunk)
        return carry

    lax.fori_loop(0, nfull, outer, 0)
    if ntail:
        do_rows(base + nfull * chunk, ntail)

    @pl.when(w < rem)
    def _():
        do_rows(NW * rpw + w, 1)


def _make_edge_agg(D, idxw, nbuf, chunk, rpw, rem, nfull, ntail):
    def body(u_hbm, src_hbm, dst_hbm, zero_hbm, out_hbm,
             sbuf, dbuf, rbufs, acc, gsems, ssems):
        c = lax.axis_index("c")
        s = lax.axis_index("s")
        w = c * NSUB + s
        _acc_zero(zero_hbm, acc, s)
        plsc.subcore_barrier()

        def do_rows(row0, nrows):
            pltpu.sync_copy(src_hbm.at[pl.ds(row0, nrows)],
                            sbuf.at[pl.ds(0, nrows)])
            pltpu.sync_copy(dst_hbm.at[pl.ds(row0, nrows)],
                            dbuf.at[pl.ds(0, nrows)])
            pend_g = [None] * nbuf
            pend_s = [None] * nbuf
            for k in range(nrows):
                b = k % nbuf
                if pend_s[b] is not None:
                    pend_s[b].wait()
                    pend_s[b] = None
                pend_g[b] = pltpu.async_copy(
                    u_hbm.at[sbuf.at[k]], rbufs[b], gsems[b])
                if k:
                    pb = (k - 1) % nbuf
                    pend_g[pb].wait()
                    pend_s[pb] = pltpu.async_copy(
                        rbufs[pb], acc.at[dbuf.at[k - 1]], ssems[pb],
                        add=True)
            lb = (nrows - 1) % nbuf
            pend_g[lb].wait()
            pend_s[lb] = pltpu.async_copy(
                rbufs[lb], acc.at[dbuf.at[nrows - 1]], ssems[lb], add=True)
            for b in range(nbuf):
                if pend_s[b] is not None:
                    pend_s[b].wait()

        _edge_sweep(w, do_rows, rpw, rem, chunk, nfull, ntail)
        plsc.subcore_barrier()
        _acc_writeback(acc, out_hbm, c, s)

    return pl.kernel(
        body,
        out_type=jax.ShapeDtypeStruct((NCORES, N, D), jnp.float32),
        mesh=_mesh(),
        compiler_params=_cparams(),
        scratch_types=[
            pltpu.VMEM((chunk, idxw), jnp.int32),
            pltpu.VMEM((chunk, idxw), jnp.int32),
            tuple(pltpu.VMEM((idxw, D), jnp.float32) for _ in range(nbuf)),
            pltpu.VMEM_SHARED((N, D), jnp.float32),
            tuple(pltpu.SemaphoreType.DMA for _ in range(nbuf)),
            tuple(pltpu.SemaphoreType.DMA for _ in range(nbuf)),
        ],
    )


_EDGE_AGG_CACHE = {}


def _agg8(u, src2d, dst2d, zeros):
    if 8 not in _EDGE_AGG_CACHE:
        _EDGE_AGG_CACHE[8] = _make_edge_agg(
            8, IDXW, NBUF, CHUNK, RPW, REM, NFULL, NTAIL)
    return _EDGE_AGG_CACHE[8](u, src2d, dst2d, zeros)


def _agg32(u, src2d, dst2d, zeros):
    if 32 not in _EDGE_AGG_CACHE:
        _EDGE_AGG_CACHE[32] = _make_edge_agg(
            32, IDXW32, NBUF32, CHUNK32, RPW32, REM32, NFULL32, NTAIL32)
    return _EDGE_AGG_CACHE[32](u, src2d, dst2d, zeros)


def _scalar_agg_body(u_hbm, src_hbm, dst_hbm, zero_hbm, out_hbm,
                     sbuf, dbuf, gtab, ubufs, acc, ssems):
    c = lax.axis_index("c")
    s = lax.axis_index("s")
    w = c * NSUB + s
    _acc_zero(zero_hbm, acc, s)
    pltpu.sync_copy(u_hbm, gtab)
    plsc.subcore_barrier()

    def do_rows(row0, nrows):
        pltpu.sync_copy(src_hbm.at[pl.ds(row0, nrows)],
                        sbuf.at[pl.ds(0, nrows)])
        pltpu.sync_copy(dst_hbm.at[pl.ds(row0, nrows)],
                        dbuf.at[pl.ds(0, nrows)])
        pend_s = [None, None]
        for k in range(nrows):
            b = k & 1
            if pend_s[b] is not None:
                pend_s[b].wait()
                pend_s[b] = None
            for l in range(IDXW // 16):
                idx = sbuf[k, pl.ds(l * 16, 16)]
                ubufs[b][pl.ds(l * 16, 16)] = plsc.load_gather(gtab, [idx])
            pend_s[b] = pltpu.async_copy(
                ubufs[b], acc.at[dbuf.at[k]], ssems[b], add=True)
        for b in (0, 1):
            if pend_s[b] is not None:
                pend_s[b].wait()

    _edge_sweep(w, do_rows)
    plsc.subcore_barrier()
    _acc_writeback(acc, out_hbm, c, s)


def _agg_scalar(u, src2d, dst2d, zeros):
    if "scalar" not in _EDGE_AGG_CACHE:
        _EDGE_AGG_CACHE["scalar"] = pl.kernel(
            _scalar_agg_body,
            out_type=jax.ShapeDtypeStruct((NCORES, N), jnp.float32),
            mesh=_mesh(),
            compiler_params=_cparams(),
            scratch_types=[
                pltpu.VMEM((CHUNK, IDXW), jnp.int32),
                pltpu.VMEM((CHUNK, IDXW), jnp.int32),
                pltpu.VMEM((N,), jnp.float32),
                (pltpu.VMEM((IDXW,), jnp.float32),
                 pltpu.VMEM((IDXW,), jnp.float32)),
                pltpu.VMEM_SHARED((N,), jnp.float32),
                (pltpu.SemaphoreType.DMA, pltpu.SemaphoreType.DMA),
            ],
        )
    return _EDGE_AGG_CACHE["scalar"](u, src2d, dst2d, zeros)


def _edge_count_body(dst_hbm, zero_hbm, out_hbm, dbuf, ones, acc):
    c = lax.axis_index("c")
    s = lax.axis_index("s")
    w = c * NSUB + s
    _acc_zero(zero_hbm, acc, s)

    def fill(k, carry):
        ones[pl.ds(k * 16, 16)] = jnp.ones((16,), jnp.float32)
        return carry

    lax.fori_loop(0, IDXW // 16, fill, 0)
    plsc.subcore_barrier()

    def do_rows(row0, nrows):
        pltpu.sync_copy(dst_hbm.at[pl.ds(row0, nrows)],
                        dbuf.at[pl.ds(0, nrows)])

        def inner(k, carry):
            pltpu.sync_copy(ones, acc.at[dbuf.at[k]], add=True)
            return carry

        lax.fori_loop(0, nrows, inner, 0)

    _edge_sweep(w, do_rows)
    plsc.subcore_barrier()
    _acc_writeback(acc, out_hbm, c, s)


def _edge_count(dst2d, zeros):
    if "count" not in _EDGE_AGG_CACHE:
        _EDGE_AGG_CACHE["count"] = pl.kernel(
            _edge_count_body,
            out_type=jax.ShapeDtypeStruct((NCORES, N), jnp.float32),
            mesh=_mesh(),
            compiler_params=_cparams(),
            scratch_types=[
                pltpu.VMEM((CHUNK, IDXW), jnp.int32),
                pltpu.VMEM((IDXW,), jnp.float32),
                pltpu.VMEM_SHARED((N,), jnp.float32),
            ],
        )
    return _EDGE_AGG_CACHE["count"](dst2d, zeros)


def _tc_a_body(deg_ref, x_ref, dinv_ref, u1_ref):
    deg = deg_ref[0] + deg_ref[1] + 1.0
    dinv = lax.rsqrt(deg)
    dinv_ref[...] = dinv
    u1_ref[...] = dinv * x_ref[...]


def _tc_a(deg_planes, x8):
    return pl.pallas_call(
        _tc_a_body,
        grid=(GRID,),
        in_specs=[
            pl.BlockSpec((2, BLK, 1), lambda i: (0, i, 0)),
            pl.BlockSpec((BLK, 8), lambda i: (i, 0)),
        ],
        out_specs=[
            pl.BlockSpec((BLK, 1), lambda i: (i, 0)),
            pl.BlockSpec((BLK, 8), lambda i: (i, 0)),
        ],
        out_shape=[
            jax.ShapeDtypeStruct((N, 1), jnp.float32),
            jax.ShapeDtypeStruct((N, 8), jnp.float32),
        ],
    )(deg_planes, x8)


def _tc_b_body(agg_ref, x_ref, dinv_ref, W1_ref, b1_ref, Ws_ref,
               h1_ref, s_ref, su_ref):
    dinv = dinv_ref[...]
    agg = agg_ref[0] + agg_ref[1]
    z = dinv * agg + (dinv * dinv) * x_ref[...]
    h1 = jnp.maximum(jnp.dot(z, W1_ref[...],
                             preferred_element_type=jnp.float32)
                     + b1_ref[...], 0.0)
    h1_ref[...] = h1
    s = jnp.dot(h1, Ws_ref[...], preferred_element_type=jnp.float32)
    s_ref[...] = s
    su_ref[...] = dinv * s


def _tc_b(agg_planes, x8, dinv, W1p, b1, Ws):
    return pl.pallas_call(
        _tc_b_body,
        grid=(GRID,),
        in_specs=[
            pl.BlockSpec((2, BLK, 8), lambda i: (0, i, 0)),
            pl.BlockSpec((BLK, 8), lambda i: (i, 0)),
            pl.BlockSpec((BLK, 1), lambda i: (i, 0)),
            pl.BlockSpec((8, 64), lambda i: (0, 0)),
            pl.BlockSpec((1, 64), lambda i: (0, 0)),
            pl.BlockSpec((64, 1), lambda i: (0, 0)),
        ],
        out_specs=[
            pl.BlockSpec((BLK, 64), lambda i: (i, 0)),
            pl.BlockSpec((BLK, 1), lambda i: (i, 0)),
            pl.BlockSpec((BLK, 1), lambda i: (i, 0)),
        ],
        out_shape=[
            jax.ShapeDtypeStruct((N, 64), jnp.float32),
            jax.ShapeDtypeStruct((N, 1), jnp.float32),
            jax.ShapeDtypeStruct((N, 1), jnp.float32),
        ],
    )(agg_planes, x8, dinv, W1p, b1, Ws)


def _tc_select_body(sagg_ref, s_ref, dinv_ref, bs_ref, mask_ref, gate_ref):
    dinv = dinv_ref[...]
    score = dinv * (sagg_ref[0] + sagg_ref[1]) + dinv * dinv * s_ref[...] \
        + bs_ref[0, 0]
    pos = (lax.broadcasted_iota(jnp.int32, (NROW, 128), 0) * 128
           + lax.broadcasted_iota(jnp.int32, (NROW, 128), 1))
    valid = pos < N
    bits = lax.bitcast_convert_type(score, jnp.uint32)
    key = jnp.where(bits >= jnp.uint32(0x80000000), ~bits,
                    bits | jnp.uint32(0x80000000))
    key = jnp.where(valid, key, jnp.uint32(0))

    def bit_step(i, t):
        b = jnp.uint32(31) - i.astype(jnp.uint32)
        cand = t | (jnp.uint32(1) << b)
        cnt = jnp.sum((key >= cand).astype(jnp.float32))
        return jnp.where(cnt >= K, cand, t)

    T = lax.fori_loop(0, 32, bit_step, jnp.uint32(0))
    n_gt = jnp.sum((key > T).astype(jnp.float32))
    need = K - n_gt
    eq = key == T

    def pos_step(i, xacc):
        b = 16 - i
        cand = xacc | (1 << b)
        cnt = jnp.sum(jnp.where(eq & (pos < cand), 1.0, 0.0))
        return jnp.where(cnt < need, cand, xacc)

    X = lax.fori_loop(0, 17, pos_step, 0)
    keep = (key > T) | (eq & (pos <= X) & (need >= 1.0))
    maskf = jnp.where(keep, 1.0, 0.0)
    mask_ref[...] = maskf
    gate_ref[...] = jnp.tanh(score) * maskf


def _tc_select(sagg_planes, s2d, dinv2d, bs):
    return pl.pallas_call(
        _tc_select_body,
        out_shape=[
            jax.ShapeDtypeStruct((NROW, 128), jnp.float32),
            jax.ShapeDtypeStruct((NROW, 128), jnp.float32),
        ],
    )(sagg_planes, s2d, dinv2d, bs)


def _tc_d_body(magg_ref, mask_ref, gate_ref, h1_ref, W2_ref,
               hw2_ref, dinv2_ref, u2_ref):
    hf = h1_ref[...] * gate_ref[...]
    hw2 = jnp.dot(hf, W2_ref[...], preferred_element_type=jnp.float32)
    hw2_ref[...] = hw2
    m = mask_ref[...]
    deg2 = m * (magg_ref[0] + magg_ref[1] + 1.0)
    dinv2 = jnp.where(deg2 > 0, 1.0 / jnp.sqrt(jnp.maximum(deg2, 1e-12)),
                      0.0)
    dinv2_ref[...] = dinv2
    u2_ref[...] = dinv2 * hw2


def _tc_d(magg_planes, mask, gate, h1, W2):
    return pl.pallas_call(
        _tc_d_body,
        grid=(GRID,),
        in_specs=[
            pl.BlockSpec((2, BLK, 1), lambda i: (0, i, 0)),
            pl.BlockSpec((BLK, 1), lambda i: (i, 0)),
            pl.BlockSpec((BLK, 1), lambda i: (i, 0)),
            pl.BlockSpec((BLK, 64), lambda i: (i, 0)),
            pl.BlockSpec((64, 32), lambda i: (0, 0)),
        ],
        out_specs=[
            pl.BlockSpec((BLK, 32), lambda i: (i, 0)),
            pl.BlockSpec((BLK, 1), lambda i: (i, 0)),
            pl.BlockSpec((BLK, 32), lambda i: (i, 0)),
        ],
        out_shape=[
            jax.ShapeDtypeStruct((N, 32), jnp.float32),
            jax.ShapeDtypeStruct((N, 1), jnp.float32),
            jax.ShapeDtypeStruct((N, 32), jnp.float32),
        ],
    )(magg_planes, mask, gate, h1, W2)


def _tc_ef_body(agg2_ref, hw2_ref, dinv2_ref, mask_ref, b2_ref,
                fc1W_ref, fc1b_ref, fc2W_ref, ct_ref, fc3W_ref, cs_ref,
                fc4W_ref, al_ref, fc5W_ref, cet_ref, fc2b_ref, fc3b_ref,
                fc4b_ref, fc5b_ref, o_ref, t32_acc):
    i = pl.program_id(0)

    @pl.when(i == 0)
    def _():
        t32_acc[...] = jnp.zeros((1, 32), jnp.float32)

    dinv2 = dinv2_ref[...]
    agg2 = agg2_ref[0] + agg2_ref[1]
    h2 = jnp.maximum(dinv2 * agg2 + (dinv2 * dinv2) * hw2_ref[...]
                     + b2_ref[...], 0.0)
    t32_acc[...] += jnp.sum(h2 * mask_ref[...], axis=0, keepdims=True)

    @pl.when(i == GRID - 1)
    def _():
        csum = (jnp.sum(fc2W_ref[...] * ct_ref[...]) + fc2b_ref[0, 0]
                + jnp.sum(fc3W_ref[...] * cs_ref[...]) + fc3b_ref[0, 0]
                + jnp.sum(fc4W_ref[...] * al_ref[...]) + fc4b_ref[0, 0]
                + jnp.sum(fc5W_ref[...] * cet_ref[...]) + fc5b_ref[0, 0])
        tot = jnp.sum(t32_acc[...] * fc1W_ref[...]) + K * fc1b_ref[0, 0]
        o_ref[...] = jnp.full((1, 1), tot * csum / (4.0 * K), jnp.float32)


def _tc_ef(agg2_planes, hw2, dinv2, mask, b2, fc1_W, fc1_b,
           fc2_W, ct_vec, fc3_W, cs_vec, fc4_W, al_vec, fc5_W, cet_vec,
           fc2_b, fc3_b, fc4_b, fc5_b):
    const = lambda i: (0, 0)
    return pl.pallas_call(
        _tc_ef_body,
        grid=(GRID,),
        in_specs=[
            pl.BlockSpec((2, BLK, 32), lambda i: (0, i, 0)),
            pl.BlockSpec((BLK, 32), lambda i: (i, 0)),
            pl.BlockSpec((BLK, 1), lambda i: (i, 0)),
            pl.BlockSpec((BLK, 1), lambda i: (i, 0)),
            pl.BlockSpec((1, 32), const),
            pl.BlockSpec((1, 32), const),
            pl.BlockSpec((1, 1), const),
            pl.BlockSpec((1, 33), const),
            pl.BlockSpec((1, 33), const),
            pl.BlockSpec((1, 25), const),
            pl.BlockSpec((1, 25), const),
            pl.BlockSpec((1, 52), const),
            pl.BlockSpec((1, 52), const),
            pl.BlockSpec((1, 10), const),
            pl.BlockSpec((1, 10), const),
            pl.BlockSpec((1, 1), const),
            pl.BlockSpec((1, 1), const),
            pl.BlockSpec((1, 1), const),
            pl.BlockSpec((1, 1), const),
        ],
        out_specs=pl.BlockSpec((1, 1), const),
        out_shape=jax.ShapeDtypeStruct((1, 1), jnp.float32),
        scratch_shapes=[pltpu.VMEM((1, 32), jnp.float32)],
    )(agg2_planes, hw2, dinv2, mask, b2, fc1_W, fc1_b,
      fc2_W, ct_vec, fc3_W, cs_vec, fc4_W, al_vec, fc5_W, cet_vec,
      fc2_b, fc3_b, fc4_b, fc5_b)


def _pad2d(v):
    return jnp.pad(v, (0, NPAD - N)).reshape(NROW, 128)


def kernel(x, edge_index, batch, y, W1, b1, Ws, bs, W2, b2, fc1_W, fc1_b,
           fc2_W, fc2_b, fc3_W, fc3_b, fc4_W, fc4_b, fc5_W, fc5_b,
           ct_vec, cs_vec, al_vec, cet_vec):
    src2d = edge_index[0].reshape(ROWS, IDXW)
    dst2d = edge_index[1].reshape(ROWS, IDXW)
    src2d32 = edge_index[0].reshape(ROWS32, IDXW32)
    dst2d32 = edge_index[1].reshape(ROWS32, IDXW32)
    zeros1 = jnp.zeros((N,), jnp.float32)
    zeros8 = jnp.zeros((N, 8), jnp.float32)
    zeros32 = jnp.zeros((N, 32), jnp.float32)
    x8 = jnp.pad(x, ((0, 0), (0, 2)))
    W1p = jnp.pad(W1, ((0, 2), (0, 0)))

    deg_pl = _edge_count(dst2d, zeros1)
    dinv, u1 = _tc_a(deg_pl[:, :, None], x8)

    agg1_pl = _agg8(u1, src2d, dst2d, zeros8)
    h1, s, su = _tc_b(agg1_pl, x8, dinv, W1p, b1[None, :], Ws)

    sagg_pl = _agg_scalar(su[:, 0], src2d, dst2d, zeros1)
    sagg2d = jnp.pad(sagg_pl, ((0, 0), (0, NPAD - N))).reshape(2, NROW, 128)
    mask2d, gate2d = _tc_select(sagg2d, _pad2d(s[:, 0]), _pad2d(dinv[:, 0]),
                                bs[None, :])
    mask = mask2d.reshape(NPAD)[:N, None]
    gate = gate2d.reshape(NPAD)[:N, None]

    magg_pl = _agg_scalar(mask[:, 0], src2d, dst2d, zeros1)
    hw2, dinv2, u2 = _tc_d(magg_pl[:, :, None], mask, gate, h1, W2)
    agg2_pl = _agg32(u2, src2d32, dst2d32, zeros32)

    out = _tc_ef(agg2_pl, hw2, dinv2, mask, b2[None, :], fc1_W,
                 fc1_b[None, :], fc2_W, ct_vec[None, :], fc3_W,
                 cs_vec[None, :], fc4_W, al_vec[None, :], fc5_W,
                 cet_vec[None, :], fc2_b[None, :], fc3_b[None, :],
                 fc4_b[None, :], fc5_b[None, :])
    return out[0]
